# baseline (speedup 1.0000x reference)
import functools

import jax
import jax.numpy as jnp
from jax import lax
from jax.experimental import pallas as pl
from jax.experimental.pallas import tpu as pltpu
from jax.experimental.pallas import tpu_sc as plsc

B, H, W, C = 16, 128, 128, 80
P = H * W
NGRP = 1024
K = 100
SCALE = 4.0
PIX_CAP = 256
SURV_CAP = 256
TOPK_PAD = 112
CH = 16
HHI = 0x7F800000


def _pmax_body(x_ref, o_ref):
    o_ref[...] = jnp.max(x_ref[...], axis=3)


def _pixel_max(cls_pred):
    bh = 64
    return pl.pallas_call(
        _pmax_body,
        grid=(B, H // bh),
        in_specs=[pl.BlockSpec((1, bh, W, C), lambda b, t: (b, t, 0, 0))],
        out_specs=pl.BlockSpec((1, bh, W), lambda b, t: (b, t, 0)),
        out_shape=jax.ShapeDtypeStruct((B, H, W), jnp.float32),
    )(cls_pred)


def _iota16():
    return lax.iota(jnp.int32, 16)


def _extract_f32(ref, i):
    blk = ref[pl.ds((i // 16) * 16, 16)]
    sel = jnp.where(_iota16() == (i % 16), blk, jnp.float32(-3.0))
    return jnp.max(sel)


def _extract_i32(ref, i):
    blk = ref[pl.ds((i // 16) * 16, 16)]
    sel = jnp.where(_iota16() == (i % 16), blk, jnp.int32(-2147483647))
    return jnp.max(sel)


def _pcnt(mask):
    return jnp.max(plsc.all_reduce_population_count(mask))


def _sc_decode(cls_pred, loc_pred, pmax):
    mesh = plsc.VectorSubcoreMesh(core_axis_name="c", subcore_axis_name="s")

    @functools.partial(
        pl.kernel,
        out_type=jax.ShapeDtypeStruct((B, 640), jnp.float32),
        mesh=mesh,
        compiler_params=pltpu.CompilerParams(needs_layout_passes=False),
        scratch_types=[
            pltpu.VMEM((H, W), jnp.float32),
            pltpu.VMEM((NGRP,), jnp.float32),
            pltpu.VMEM((PIX_CAP + 16,), jnp.int32),
            pltpu.VMEM((CH * 8, C), jnp.float32),
            pltpu.VMEM((SURV_CAP + 16,), jnp.float32),
            pltpu.VMEM((SURV_CAP + 16,), jnp.int32),
            pltpu.VMEM((TOPK_PAD,), jnp.int32),
            pltpu.VMEM((CH * 8, 4), jnp.float32),
            pltpu.VMEM((640,), jnp.float32),
            pltpu.SemaphoreType.DMA,
            pltpu.SemaphoreType.DMA,
        ],
    )
    def k(cls_hbm, loc_hbm, pmax_hbm, out_hbm,
          pbuf, gmax, pix, cbuf, sval, sidx, spat, lbuf, det, gsem, lsem):
        c = lax.axis_index("c")
        s = lax.axis_index("s")

        @pl.when(s < 8)
        def _work():
            b = c * 8 + s
            iota = _iota16()

            pltpu.sync_copy(pmax_hbm.at[b], pbuf)

            def gm_body(i, _):
                r0 = i // 8
                col = (i % 8) * 16

                def inner(j, acc):
                    return jnp.maximum(acc, pbuf[r0 + j * 8, pl.ds(col, 16)])

                acc = lax.fori_loop(1, 16, inner, pbuf[r0, pl.ds(col, 16)])
                gmax[pl.ds(i * 16, 16)] = acc
                return 0

            lax.fori_loop(0, NGRP // 16, gm_body, 0)

            def bs_body(_, carry):
                lo, hi = carry
                mid = lo + (hi - lo) // 2
                tf = lax.bitcast_convert_type(mid, jnp.float32)

                def cnt(g, acc):
                    return acc + plsc.all_reduce_population_count(
                        gmax[pl.ds(g * 16, 16)] >= tf)

                csplat = lax.fori_loop(0, NGRP // 16, cnt,
                                       jnp.zeros((16,), jnp.int32))
                big = jnp.max(csplat) >= K
                return (jnp.where(big, mid, lo), jnp.where(big, hi, mid))

            lo, _hi = lax.fori_loop(0, 31, bs_body,
                                    (jnp.int32(0), jnp.int32(HHI)))
            tf2 = lax.bitcast_convert_type(lo, jnp.float32)

            def cp_body(q, off):
                m = pbuf[q // 8, pl.ds((q % 8) * 16, 16)] >= tf2
                offc = jnp.minimum(off, PIX_CAP)
                plsc.store_compressed(pix.at[pl.ds(offc, 16)],
                                      q * 16 + iota, mask=m)
                return offc + _pcnt(m)

            npix = lax.fori_loop(0, P // 16, cp_body, jnp.int32(0))
            npix = jnp.minimum(npix, PIX_CAP)

            def ch_body(ci, moff):
                def issue(q, _):
                    fi = ci * CH + q

                    @pl.when(fi < npix)
                    def _():
                        p = _extract_i32(pix, fi)
                        y = p // W
                        x8 = ((p % W) // 8) * 8
                        pltpu.async_copy(
                            cls_hbm.at[b, y, pl.ds(x8, 8)],
                            cbuf.at[pl.ds(q * 8, 8)], gsem)
                    return 0

                lax.fori_loop(0, CH, issue, 0)

                def drain(q, _):
                    fi = ci * CH + q

                    @pl.when(fi < npix)
                    def _():
                        p = _extract_i32(pix, fi)
                        y = p // W
                        x8 = ((p % W) // 8) * 8
                        pltpu.make_async_copy(
                            cls_hbm.at[b, y, pl.ds(x8, 8)],
                            cbuf.at[pl.ds(q * 8, 8)], gsem).wait()
                    return 0

                lax.fori_loop(0, CH, drain, 0)

                def filt(q, moff2):
                    fi = ci * CH + q

                    def skip(m3):
                        return m3

                    def do(m3):
                        p = _extract_i32(pix, fi)
                        row = q * 8 + (p % 8)

                        def cgrp(g, m4):
                            vals = cbuf[row, pl.ds(g * 16, 16)]
                            m = vals >= tf2
                            mc = jnp.minimum(m4, SURV_CAP)
                            plsc.store_compressed(
                                sval.at[pl.ds(mc, 16)], vals, mask=m)
                            plsc.store_compressed(
                                sidx.at[pl.ds(mc, 16)],
                                p * C + g * 16 + iota, mask=m)
                            return mc + _pcnt(m)

                        return lax.fori_loop(0, C // 16, cgrp, m3)

                    return lax.cond(fi < npix, do, skip, moff2)

                return lax.fori_loop(0, CH, filt, moff)

            mcnt = lax.fori_loop(0, (npix + CH - 1) // CH, ch_body,
                                 jnp.int32(0))
            mcnt = jnp.minimum(mcnt, SURV_CAP)

            sval[pl.ds(mcnt, 16)] = jnp.full((16,), -1.0, jnp.float32)
            sidx[pl.ds(mcnt, 16)] = jnp.zeros((16,), jnp.int32)

            spat[pl.ds(96, 16)] = jnp.zeros((16,), jnp.int32)

            nblk = (mcnt + 15) // 16

            def rank_body(i, _):
                vi = _extract_f32(sval, i)
                xi = _extract_i32(sidx, i)

                def inner(g, acc):
                    vj = sval[pl.ds(g * 16, 16)]
                    xj = sidx[pl.ds(g * 16, 16)]
                    m = (vj > vi) | ((vj == vi) & (xj < xi))
                    return acc + plsc.all_reduce_population_count(m)

                rank = jnp.max(lax.fori_loop(0, nblk, inner,
                                             jnp.zeros((16,), jnp.int32)))

                @pl.when(rank < K)
                def _():
                    lane0 = iota == 0
                    plsc.store_scatter(
                        det, [jnp.full((16,), rank * 6 + 4, jnp.int32)],
                        jnp.full((16,), vi, jnp.float32), mask=lane0)
                    plsc.store_scatter(
                        det, [jnp.full((16,), rank * 6 + 5, jnp.int32)],
                        jnp.full((16,), (xi % C).astype(jnp.float32),
                                 jnp.float32), mask=lane0)
                    plsc.store_scatter(
                        spat, [jnp.full((16,), rank, jnp.int32)],
                        jnp.full((16,), xi // C, jnp.int32), mask=lane0)
                return 0

            lax.fori_loop(0, mcnt, rank_body, 0)

            def bx_body(t, _):
                def issue(q, _):
                    p = _extract_i32(spat, t * 16 + q)
                    y = p // W
                    x8 = ((p % W) // 8) * 8
                    pltpu.async_copy(loc_hbm.at[b, y, pl.ds(x8, 8)],
                                     lbuf.at[pl.ds(q * 8, 8)], lsem)
                    return 0

                lax.fori_loop(0, 16, issue, 0)

                def drain(q, _):
                    p = _extract_i32(spat, t * 16 + q)
                    y = p // W
                    x8 = ((p % W) // 8) * 8
                    pltpu.make_async_copy(
                        loc_hbm.at[b, y, pl.ds(x8, 8)],
                        lbuf.at[pl.ds(q * 8, 8)], lsem).wait()
                    return 0

                lax.fori_loop(0, 16, drain, 0)

                rows = iota + t * 16
                mrow = rows < K
                p = spat[pl.ds(t * 16, 16)]
                lrow = iota * 8 + (p % 8)
                for comp in range(4):
                    vals = plsc.load_gather(
                        lbuf, [lrow, jnp.full((16,), comp, jnp.int32)],
                        mask=mrow) * SCALE
                    plsc.store_scatter(det, [rows * 6 + comp], vals,
                                       mask=mrow)
                return 0

            lax.fori_loop(0, TOPK_PAD // 16, bx_body, 0)

            pltpu.sync_copy(det, out_hbm.at[b])

    return k(cls_pred, loc_pred, pmax)


def kernel(cls_pred, loc_pred):
    pmax = _pixel_max(cls_pred)
    det = _sc_decode(cls_pred, loc_pred, pmax)
    return det[:, :600].reshape(B, K, 6)


# speedup vs baseline: 1.0042x; 1.0042x over previous
import functools

import jax
import jax.numpy as jnp
from jax import lax
from jax.experimental import pallas as pl
from jax.experimental.pallas import tpu as pltpu
from jax.experimental.pallas import tpu_sc as plsc

B, H, W, C = 16, 128, 128, 80
P = H * W
NGRP = 1024
K = 100
SCALE = 4.0
PIX_CAP = 256
SURV_CAP = 256
TOPK_PAD = 112
CH = 32
HHI = 0x7F800000


def _pmax_body(x_ref, o_ref):
    o_ref[...] = jnp.max(x_ref[...], axis=3)


def _pixel_max(cls_pred):
    bh = 64
    return pl.pallas_call(
        _pmax_body,
        grid=(B, H // bh),
        in_specs=[pl.BlockSpec((1, bh, W, C), lambda b, t: (b, t, 0, 0))],
        out_specs=pl.BlockSpec((1, bh, W), lambda b, t: (b, t, 0)),
        out_shape=jax.ShapeDtypeStruct((B, H, W), jnp.float32),
    )(cls_pred)


def _iota16():
    return lax.iota(jnp.int32, 16)


def _extract_f32(ref, i):
    blk = ref[pl.ds((i // 16) * 16, 16)]
    sel = jnp.where(_iota16() == (i % 16), blk, jnp.float32(-3.0))
    return jnp.max(sel)


def _extract_i32(ref, i):
    blk = ref[pl.ds((i // 16) * 16, 16)]
    sel = jnp.where(_iota16() == (i % 16), blk, jnp.int32(-2147483647))
    return jnp.max(sel)


def _pcnt(mask):
    return jnp.max(plsc.all_reduce_population_count(mask))


def _sc_decode(cls_pred, loc_pred, pmax):
    mesh = plsc.VectorSubcoreMesh(core_axis_name="c", subcore_axis_name="s")

    @functools.partial(
        pl.kernel,
        out_type=jax.ShapeDtypeStruct((B, 640), jnp.float32),
        mesh=mesh,
        compiler_params=pltpu.CompilerParams(needs_layout_passes=False),
        scratch_types=[
            pltpu.VMEM((H, W), jnp.float32),
            pltpu.VMEM((NGRP,), jnp.float32),
            pltpu.VMEM((PIX_CAP + 16,), jnp.int32),
            pltpu.VMEM((CH * 8, C), jnp.float32),
            pltpu.VMEM((SURV_CAP + 16,), jnp.float32),
            pltpu.VMEM((SURV_CAP + 16,), jnp.int32),
            pltpu.VMEM((TOPK_PAD,), jnp.int32),
            pltpu.VMEM((CH * 8, 4), jnp.float32),
            pltpu.VMEM((640,), jnp.float32),
            pltpu.SemaphoreType.DMA,
            pltpu.SemaphoreType.DMA,
        ],
    )
    def k(cls_hbm, loc_hbm, pmax_hbm, out_hbm,
          pbuf, gmax, pix, cbuf, sval, sidx, spat, lbuf, det, gsem, lsem):
        c = lax.axis_index("c")
        s = lax.axis_index("s")

        @pl.when(s < 8)
        def _work():
            b = c * 8 + s
            iota = _iota16()

            pltpu.sync_copy(pmax_hbm.at[b], pbuf)

            def gm_body(i, _):
                r0 = i // 8
                col = (i % 8) * 16

                def inner(j, acc):
                    return jnp.maximum(acc, pbuf[r0 + j * 8, pl.ds(col, 16)])

                acc = lax.fori_loop(1, 16, inner, pbuf[r0, pl.ds(col, 16)])
                gmax[pl.ds(i * 16, 16)] = acc
                return 0

            lax.fori_loop(0, NGRP // 16, gm_body, 0)

            def bs_body(_, carry):
                lo, hi = carry
                mid = lo + (hi - lo) // 2
                tf = lax.bitcast_convert_type(mid, jnp.float32)

                def cnt(g, acc):
                    return acc + plsc.all_reduce_population_count(
                        gmax[pl.ds(g * 16, 16)] >= tf)

                csplat = lax.fori_loop(0, NGRP // 16, cnt,
                                       jnp.zeros((16,), jnp.int32))
                big = jnp.max(csplat) >= K
                return (jnp.where(big, mid, lo), jnp.where(big, hi, mid))

            lo, _hi = lax.fori_loop(0, 31, bs_body,
                                    (jnp.int32(0), jnp.int32(HHI)))
            tf2 = lax.bitcast_convert_type(lo, jnp.float32)

            def cp_body(q, off):
                m = pbuf[q // 8, pl.ds((q % 8) * 16, 16)] >= tf2
                offc = jnp.minimum(off, PIX_CAP)
                plsc.store_compressed(pix.at[pl.ds(offc, 16)],
                                      q * 16 + iota, mask=m)
                return offc + _pcnt(m)

            npix = lax.fori_loop(0, P // 16, cp_body, jnp.int32(0))
            npix = jnp.minimum(npix, PIX_CAP)

            def ch_body(ci, moff):
                def issue(q, _):
                    fi = ci * CH + q

                    @pl.when(fi < npix)
                    def _():
                        p = _extract_i32(pix, fi)
                        y = p // W
                        x8 = ((p % W) // 8) * 8
                        pltpu.async_copy(
                            cls_hbm.at[b, y, pl.ds(x8, 8)],
                            cbuf.at[pl.ds(q * 8, 8)], gsem)
                    return 0

                lax.fori_loop(0, CH, issue, 0)

                def drain(q, _):
                    fi = ci * CH + q

                    @pl.when(fi < npix)
                    def _():
                        p = _extract_i32(pix, fi)
                        y = p // W
                        x8 = ((p % W) // 8) * 8
                        pltpu.make_async_copy(
                            cls_hbm.at[b, y, pl.ds(x8, 8)],
                            cbuf.at[pl.ds(q * 8, 8)], gsem).wait()
                    return 0

                lax.fori_loop(0, CH, drain, 0)

                def filt(q, moff2):
                    fi = ci * CH + q

                    def skip(m3):
                        return m3

                    def do(m3):
                        p = _extract_i32(pix, fi)
                        row = q * 8 + (p % 8)

                        def cgrp(g, m4):
                            vals = cbuf[row, pl.ds(g * 16, 16)]
                            m = vals >= tf2
                            mc = jnp.minimum(m4, SURV_CAP)
                            plsc.store_compressed(
                                sval.at[pl.ds(mc, 16)], vals, mask=m)
                            plsc.store_compressed(
                                sidx.at[pl.ds(mc, 16)],
                                p * C + g * 16 + iota, mask=m)
                            return mc + _pcnt(m)

                        return lax.fori_loop(0, C // 16, cgrp, m3)

                    return lax.cond(fi < npix, do, skip, moff2)

                return lax.fori_loop(0, CH, filt, moff)

            mcnt = lax.fori_loop(0, (npix + CH - 1) // CH, ch_body,
                                 jnp.int32(0))
            mcnt = jnp.minimum(mcnt, SURV_CAP)

            sval[pl.ds(mcnt, 16)] = jnp.full((16,), -1.0, jnp.float32)
            sidx[pl.ds(mcnt, 16)] = jnp.zeros((16,), jnp.int32)

            spat[pl.ds(96, 16)] = jnp.zeros((16,), jnp.int32)

            nblk = (mcnt + 15) // 16

            def rank_body(i, _):
                vi = _extract_f32(sval, i)
                xi = _extract_i32(sidx, i)

                def inner(g, acc):
                    vj = sval[pl.ds(g * 16, 16)]
                    xj = sidx[pl.ds(g * 16, 16)]
                    m = (vj > vi) | ((vj == vi) & (xj < xi))
                    return acc + plsc.all_reduce_population_count(m)

                rank = jnp.max(lax.fori_loop(0, nblk, inner,
                                             jnp.zeros((16,), jnp.int32)))

                @pl.when(rank < K)
                def _():
                    lane0 = iota == 0
                    plsc.store_scatter(
                        det, [jnp.full((16,), rank * 6 + 4, jnp.int32)],
                        jnp.full((16,), vi, jnp.float32), mask=lane0)
                    plsc.store_scatter(
                        det, [jnp.full((16,), rank * 6 + 5, jnp.int32)],
                        jnp.full((16,), (xi % C).astype(jnp.float32),
                                 jnp.float32), mask=lane0)
                    plsc.store_scatter(
                        spat, [jnp.full((16,), rank, jnp.int32)],
                        jnp.full((16,), xi // C, jnp.int32), mask=lane0)
                return 0

            lax.fori_loop(0, mcnt, rank_body, 0)

            def bx_body(t, _):
                def issue(q, _):
                    p = _extract_i32(spat, t * 16 + q)
                    y = p // W
                    x8 = ((p % W) // 8) * 8
                    pltpu.async_copy(loc_hbm.at[b, y, pl.ds(x8, 8)],
                                     lbuf.at[pl.ds(q * 8, 8)], lsem)
                    return 0

                lax.fori_loop(0, 16, issue, 0)

                def drain(q, _):
                    p = _extract_i32(spat, t * 16 + q)
                    y = p // W
                    x8 = ((p % W) // 8) * 8
                    pltpu.make_async_copy(
                        loc_hbm.at[b, y, pl.ds(x8, 8)],
                        lbuf.at[pl.ds(q * 8, 8)], lsem).wait()
                    return 0

                lax.fori_loop(0, 16, drain, 0)

                rows = iota + t * 16
                mrow = rows < K
                p = spat[pl.ds(t * 16, 16)]
                lrow = iota * 8 + (p % 8)
                for comp in range(4):
                    vals = plsc.load_gather(
                        lbuf, [lrow, jnp.full((16,), comp, jnp.int32)],
                        mask=mrow) * SCALE
                    plsc.store_scatter(det, [rows * 6 + comp], vals,
                                       mask=mrow)
                return 0

            lax.fori_loop(0, TOPK_PAD // 16, bx_body, 0)

            pltpu.sync_copy(det, out_hbm.at[b])

    return k(cls_pred, loc_pred, pmax)


def kernel(cls_pred, loc_pred):
    pmax = _pixel_max(cls_pred)
    det = _sc_decode(cls_pred, loc_pred, pmax)
    return det[:, :600].reshape(B, K, 6)


# speedup vs baseline: 1.0707x; 1.0662x over previous
import functools

import jax
import jax.numpy as jnp
from jax import lax
from jax.experimental import pallas as pl
from jax.experimental.pallas import tpu as pltpu
from jax.experimental.pallas import tpu_sc as plsc

B, H, W, C = 16, 128, 128, 80
P = H * W
NGRP = 1024
K = 100
SCALE = 4.0
PIX_CAP = 256
SURV_CAP = 256
TOPK_PAD = 112
CH = 32
HHI = 0x7F800000


def _pmax_body(x_ref, o_ref):
    o_ref[...] = jnp.max(x_ref[...], axis=3)


def _pixel_max(cls_pred):
    bh = 64
    return pl.pallas_call(
        _pmax_body,
        grid=(B, H // bh),
        in_specs=[pl.BlockSpec((1, bh, W, C), lambda b, t: (b, t, 0, 0))],
        out_specs=pl.BlockSpec((1, bh, W), lambda b, t: (b, t, 0)),
        out_shape=jax.ShapeDtypeStruct((B, H, W), jnp.float32),
    )(cls_pred)


def _iota16():
    return lax.iota(jnp.int32, 16)


def _extract_f32(ref, i):
    blk = ref[pl.ds((i // 16) * 16, 16)]
    sel = jnp.where(_iota16() == (i % 16), blk, jnp.float32(-3.0))
    return jnp.max(sel)


def _extract_i32(ref, i):
    blk = ref[pl.ds((i // 16) * 16, 16)]
    sel = jnp.where(_iota16() == (i % 16), blk, jnp.int32(-2147483647))
    return jnp.max(sel)


def _pcnt(mask):
    return jnp.max(plsc.all_reduce_population_count(mask))


def _sc_decode(cls_pred, loc_blocks, pmax):
    mesh = plsc.VectorSubcoreMesh(core_axis_name="c", subcore_axis_name="s")

    @functools.partial(
        pl.kernel,
        out_type=jax.ShapeDtypeStruct((B, 640), jnp.float32),
        mesh=mesh,
        compiler_params=pltpu.CompilerParams(needs_layout_passes=False),
        scratch_types=[
            pltpu.VMEM((H, W), jnp.float32),
            pltpu.VMEM((NGRP,), jnp.float32),
            pltpu.VMEM((PIX_CAP + 16,), jnp.int32),
            pltpu.VMEM((CH * 8, C), jnp.float32),
            pltpu.VMEM((SURV_CAP + 16,), jnp.float32),
            pltpu.VMEM((SURV_CAP + 16,), jnp.int32),
            pltpu.VMEM((TOPK_PAD,), jnp.int32),
            pltpu.VMEM((16, 128), jnp.float32),
            pltpu.VMEM((640,), jnp.float32),
            pltpu.SemaphoreType.DMA,
            pltpu.SemaphoreType.DMA,
        ],
    )
    def k(cls_hbm, loc_hbm, pmax_hbm, out_hbm,
          pbuf, gmax, pix, cbuf, sval, sidx, spat, lbuf, det, gsem, lsem):
        c = lax.axis_index("c")
        s = lax.axis_index("s")

        @pl.when(s < 8)
        def _work():
            b = c * 8 + s
            iota = _iota16()

            pltpu.sync_copy(pmax_hbm.at[b], pbuf)

            def gm_body(i, _):
                r0 = i // 8
                col = (i % 8) * 16

                def inner(j, acc):
                    return jnp.maximum(acc, pbuf[r0 + j * 8, pl.ds(col, 16)])

                acc = lax.fori_loop(1, 16, inner, pbuf[r0, pl.ds(col, 16)])
                gmax[pl.ds(i * 16, 16)] = acc
                return 0

            lax.fori_loop(0, NGRP // 16, gm_body, 0)

            def bs_body(_, carry):
                lo, hi = carry
                mid = lo + (hi - lo) // 2
                tf = lax.bitcast_convert_type(mid, jnp.float32)

                def cnt(g, acc):
                    return acc + plsc.all_reduce_population_count(
                        gmax[pl.ds(g * 16, 16)] >= tf)

                csplat = lax.fori_loop(0, NGRP // 16, cnt,
                                       jnp.zeros((16,), jnp.int32))
                big = jnp.max(csplat) >= K
                return (jnp.where(big, mid, lo), jnp.where(big, hi, mid))

            lo, _hi = lax.fori_loop(0, 31, bs_body,
                                    (jnp.int32(0), jnp.int32(HHI)))
            tf2 = lax.bitcast_convert_type(lo, jnp.float32)

            def cp_body(q, off):
                m = pbuf[q // 8, pl.ds((q % 8) * 16, 16)] >= tf2
                offc = jnp.minimum(off, PIX_CAP)
                plsc.store_compressed(pix.at[pl.ds(offc, 16)],
                                      q * 16 + iota, mask=m)
                return offc + _pcnt(m)

            npix = lax.fori_loop(0, P // 16, cp_body, jnp.int32(0))
            npix = jnp.minimum(npix, PIX_CAP)

            def ch_body(ci, moff):
                def issue(q, _):
                    fi = ci * CH + q

                    @pl.when(fi < npix)
                    def _():
                        p = _extract_i32(pix, fi)
                        y = p // W
                        x8 = ((p % W) // 8) * 8
                        pltpu.async_copy(
                            cls_hbm.at[b, y, pl.ds(x8, 8)],
                            cbuf.at[pl.ds(q * 8, 8)], gsem)
                    return 0

                lax.fori_loop(0, CH, issue, 0)

                def drain(q, _):
                    fi = ci * CH + q

                    @pl.when(fi < npix)
                    def _():
                        p = _extract_i32(pix, fi)
                        y = p // W
                        x8 = ((p % W) // 8) * 8
                        pltpu.make_async_copy(
                            cls_hbm.at[b, y, pl.ds(x8, 8)],
                            cbuf.at[pl.ds(q * 8, 8)], gsem).wait()
                    return 0

                lax.fori_loop(0, CH, drain, 0)

                def filt(q, moff2):
                    fi = ci * CH + q

                    def skip(m3):
                        return m3

                    def do(m3):
                        p = _extract_i32(pix, fi)
                        row = q * 8 + (p % 8)

                        def cgrp(g, m4):
                            vals = cbuf[row, pl.ds(g * 16, 16)]
                            m = vals >= tf2
                            mc = jnp.minimum(m4, SURV_CAP)
                            plsc.store_compressed(
                                sval.at[pl.ds(mc, 16)], vals, mask=m)
                            plsc.store_compressed(
                                sidx.at[pl.ds(mc, 16)],
                                p * C + g * 16 + iota, mask=m)
                            return mc + _pcnt(m)

                        return lax.fori_loop(0, C // 16, cgrp, m3)

                    return lax.cond(fi < npix, do, skip, moff2)

                return lax.fori_loop(0, CH, filt, moff)

            mcnt = lax.fori_loop(0, (npix + CH - 1) // CH, ch_body,
                                 jnp.int32(0))
            mcnt = jnp.minimum(mcnt, SURV_CAP)

            sval[pl.ds(mcnt, 16)] = jnp.full((16,), -1.0, jnp.float32)
            sidx[pl.ds(mcnt, 16)] = jnp.zeros((16,), jnp.int32)

            spat[pl.ds(96, 16)] = jnp.zeros((16,), jnp.int32)

            nblk = (mcnt + 15) // 16

            def rank_body(i, _):
                vi = _extract_f32(sval, i)
                xi = _extract_i32(sidx, i)

                def inner(g, acc):
                    vj = sval[pl.ds(g * 16, 16)]
                    xj = sidx[pl.ds(g * 16, 16)]
                    m = (vj > vi) | ((vj == vi) & (xj < xi))
                    return acc + plsc.all_reduce_population_count(m)

                rank = jnp.max(lax.fori_loop(0, nblk, inner,
                                             jnp.zeros((16,), jnp.int32)))

                @pl.when(rank < K)
                def _():
                    lane0 = iota == 0
                    plsc.store_scatter(
                        det, [jnp.full((16,), rank * 6 + 4, jnp.int32)],
                        jnp.full((16,), vi, jnp.float32), mask=lane0)
                    plsc.store_scatter(
                        det, [jnp.full((16,), rank * 6 + 5, jnp.int32)],
                        jnp.full((16,), (xi % C).astype(jnp.float32),
                                 jnp.float32), mask=lane0)
                    plsc.store_scatter(
                        spat, [jnp.full((16,), rank, jnp.int32)],
                        jnp.full((16,), xi // C, jnp.int32), mask=lane0)
                return 0

            lax.fori_loop(0, mcnt, rank_body, 0)

            def bx_body(t, _):
                def issue(q, _):
                    p = _extract_i32(spat, t * 16 + q)
                    pltpu.async_copy(loc_hbm.at[b, p // 32],
                                     lbuf.at[q], lsem)
                    return 0

                lax.fori_loop(0, 16, issue, 0)

                def drain(q, _):
                    p = _extract_i32(spat, t * 16 + q)
                    pltpu.make_async_copy(loc_hbm.at[b, p // 32],
                                          lbuf.at[q], lsem).wait()
                    return 0

                lax.fori_loop(0, 16, drain, 0)

                rows = iota + t * 16
                mrow = rows < K
                p = spat[pl.ds(t * 16, 16)]
                base = (p % 32) * 4
                for comp in range(4):
                    vals = plsc.load_gather(
                        lbuf, [iota, base + comp], mask=mrow) * SCALE
                    plsc.store_scatter(det, [rows * 6 + comp], vals,
                                       mask=mrow)
                return 0

            lax.fori_loop(0, TOPK_PAD // 16, bx_body, 0)

            pltpu.sync_copy(det, out_hbm.at[b])

    return k(cls_pred, loc_blocks, pmax)


def kernel(cls_pred, loc_pred):
    pmax = _pixel_max(cls_pred)
    loc_blocks = loc_pred.reshape(B, H * W * 4 // 128, 128)
    det = _sc_decode(cls_pred, loc_blocks, pmax)
    return det[:, :600].reshape(B, K, 6)


# speedup vs baseline: 1.1052x; 1.0322x over previous
import functools

import jax
import jax.numpy as jnp
from jax import lax
from jax.experimental import pallas as pl
from jax.experimental.pallas import tpu as pltpu
from jax.experimental.pallas import tpu_sc as plsc

B, H, W, C = 16, 128, 128, 80
P = H * W
NGRP = 1024
K = 100
SCALE = 4.0
PIX_CAP = 256
SURV_CAP = 256
TOPK_PAD = 112
CH = 32
HHI = 0x7F800000


def _pmax_body(x_ref, o_ref):
    o_ref[...] = jnp.max(x_ref[...], axis=3)


def _pixel_max(cls_pred):
    bh = 128
    return pl.pallas_call(
        _pmax_body,
        grid=(B, H // bh),
        in_specs=[pl.BlockSpec((1, bh, W, C), lambda b, t: (b, t, 0, 0))],
        out_specs=pl.BlockSpec((1, bh, W), lambda b, t: (b, t, 0)),
        out_shape=jax.ShapeDtypeStruct((B, H, W), jnp.float32),
    )(cls_pred)


def _iota16():
    return lax.iota(jnp.int32, 16)


def _extract_f32(ref, i):
    blk = ref[pl.ds((i // 16) * 16, 16)]
    sel = jnp.where(_iota16() == (i % 16), blk, jnp.float32(-3.0))
    return jnp.max(sel)


def _extract_i32(ref, i):
    blk = ref[pl.ds((i // 16) * 16, 16)]
    sel = jnp.where(_iota16() == (i % 16), blk, jnp.int32(-2147483647))
    return jnp.max(sel)


def _pcnt(mask):
    return jnp.max(plsc.all_reduce_population_count(mask))


def _sc_decode(cls_pred, loc_blocks, pmax):
    mesh = plsc.VectorSubcoreMesh(core_axis_name="c", subcore_axis_name="s")

    @functools.partial(
        pl.kernel,
        out_type=jax.ShapeDtypeStruct((B, 640), jnp.float32),
        mesh=mesh,
        compiler_params=pltpu.CompilerParams(needs_layout_passes=False),
        scratch_types=[
            pltpu.VMEM((H, W), jnp.float32),
            pltpu.VMEM((NGRP,), jnp.float32),
            pltpu.VMEM((PIX_CAP + 16,), jnp.int32),
            pltpu.VMEM((CH * 8, C), jnp.float32),
            pltpu.VMEM((SURV_CAP + 16,), jnp.float32),
            pltpu.VMEM((SURV_CAP + 16,), jnp.int32),
            pltpu.VMEM((TOPK_PAD,), jnp.int32),
            pltpu.VMEM((16, 128), jnp.float32),
            pltpu.VMEM((640,), jnp.float32),
            pltpu.SemaphoreType.DMA,
            pltpu.SemaphoreType.DMA,
        ],
    )
    def k(cls_hbm, loc_hbm, pmax_hbm, out_hbm,
          pbuf, gmax, pix, cbuf, sval, sidx, spat, lbuf, det, gsem, lsem):
        c = lax.axis_index("c")
        s = lax.axis_index("s")

        @pl.when(s < 8)
        def _work():
            b = c * 8 + s
            iota = _iota16()

            pltpu.sync_copy(pmax_hbm.at[b], pbuf)

            def gm_body(i, _):
                r0 = i // 8
                col = (i % 8) * 16

                def inner(j, acc):
                    return jnp.maximum(acc, pbuf[r0 + j * 8, pl.ds(col, 16)])

                acc = lax.fori_loop(1, 16, inner, pbuf[r0, pl.ds(col, 16)])
                gmax[pl.ds(i * 16, 16)] = acc
                return 0

            lax.fori_loop(0, NGRP // 16, gm_body, 0)

            def bs_body(_, carry):
                lo, hi = carry
                mid = lo + (hi - lo) // 2
                tf = lax.bitcast_convert_type(mid, jnp.float32)

                def cnt(g, acc):
                    return acc + plsc.all_reduce_population_count(
                        gmax[pl.ds(g * 16, 16)] >= tf)

                csplat = lax.fori_loop(0, NGRP // 16, cnt,
                                       jnp.zeros((16,), jnp.int32))
                big = jnp.max(csplat) >= K
                return (jnp.where(big, mid, lo), jnp.where(big, hi, mid))

            lo, _hi = lax.fori_loop(0, 31, bs_body,
                                    (jnp.int32(0), jnp.int32(HHI)))
            tf2 = lax.bitcast_convert_type(lo, jnp.float32)

            def cp_body(q, off):
                m = pbuf[q // 8, pl.ds((q % 8) * 16, 16)] >= tf2
                offc = jnp.minimum(off, PIX_CAP)
                plsc.store_compressed(pix.at[pl.ds(offc, 16)],
                                      q * 16 + iota, mask=m)
                return offc + _pcnt(m)

            npix = lax.fori_loop(0, P // 16, cp_body, jnp.int32(0))
            npix = jnp.minimum(npix, PIX_CAP)

            def ch_body(ci, moff):
                def issue(q, _):
                    fi = ci * CH + q

                    @pl.when(fi < npix)
                    def _():
                        p = _extract_i32(pix, fi)
                        y = p // W
                        x8 = ((p % W) // 8) * 8
                        pltpu.async_copy(
                            cls_hbm.at[b, y, pl.ds(x8, 8)],
                            cbuf.at[pl.ds(q * 8, 8)], gsem)
                    return 0

                lax.fori_loop(0, CH, issue, 0)

                def drain(q, _):
                    fi = ci * CH + q

                    @pl.when(fi < npix)
                    def _():
                        p = _extract_i32(pix, fi)
                        y = p // W
                        x8 = ((p % W) // 8) * 8
                        pltpu.make_async_copy(
                            cls_hbm.at[b, y, pl.ds(x8, 8)],
                            cbuf.at[pl.ds(q * 8, 8)], gsem).wait()
                    return 0

                lax.fori_loop(0, CH, drain, 0)

                def filt(q, moff2):
                    fi = ci * CH + q

                    def skip(m3):
                        return m3

                    def do(m3):
                        p = _extract_i32(pix, fi)
                        row = q * 8 + (p % 8)

                        def cgrp(g, m4):
                            vals = cbuf[row, pl.ds(g * 16, 16)]
                            m = vals >= tf2
                            mc = jnp.minimum(m4, SURV_CAP)
                            plsc.store_compressed(
                                sval.at[pl.ds(mc, 16)], vals, mask=m)
                            plsc.store_compressed(
                                sidx.at[pl.ds(mc, 16)],
                                p * C + g * 16 + iota, mask=m)
                            return mc + _pcnt(m)

                        return lax.fori_loop(0, C // 16, cgrp, m3)

                    return lax.cond(fi < npix, do, skip, moff2)

                return lax.fori_loop(0, CH, filt, moff)

            mcnt = lax.fori_loop(0, (npix + CH - 1) // CH, ch_body,
                                 jnp.int32(0))
            mcnt = jnp.minimum(mcnt, SURV_CAP)

            sval[pl.ds(mcnt, 16)] = jnp.full((16,), -1.0, jnp.float32)
            sidx[pl.ds(mcnt, 16)] = jnp.zeros((16,), jnp.int32)

            spat[pl.ds(96, 16)] = jnp.zeros((16,), jnp.int32)

            nblk = (mcnt + 15) // 16

            def rank_body(i, _):
                vi = _extract_f32(sval, i)
                xi = _extract_i32(sidx, i)

                def inner(g, acc):
                    vj = sval[pl.ds(g * 16, 16)]
                    xj = sidx[pl.ds(g * 16, 16)]
                    m = (vj > vi) | ((vj == vi) & (xj < xi))
                    return acc + plsc.all_reduce_population_count(m)

                rank = jnp.max(lax.fori_loop(0, nblk, inner,
                                             jnp.zeros((16,), jnp.int32)))

                @pl.when(rank < K)
                def _():
                    lane0 = iota == 0
                    plsc.store_scatter(
                        det, [jnp.full((16,), rank * 6 + 4, jnp.int32)],
                        jnp.full((16,), vi, jnp.float32), mask=lane0)
                    plsc.store_scatter(
                        det, [jnp.full((16,), rank * 6 + 5, jnp.int32)],
                        jnp.full((16,), (xi % C).astype(jnp.float32),
                                 jnp.float32), mask=lane0)
                    plsc.store_scatter(
                        spat, [jnp.full((16,), rank, jnp.int32)],
                        jnp.full((16,), xi // C, jnp.int32), mask=lane0)
                return 0

            lax.fori_loop(0, mcnt, rank_body, 0)

            def bx_body(t, _):
                def issue(q, _):
                    p = _extract_i32(spat, t * 16 + q)
                    pltpu.async_copy(loc_hbm.at[b, p // 32],
                                     lbuf.at[q], lsem)
                    return 0

                lax.fori_loop(0, 16, issue, 0)

                def drain(q, _):
                    p = _extract_i32(spat, t * 16 + q)
                    pltpu.make_async_copy(loc_hbm.at[b, p // 32],
                                          lbuf.at[q], lsem).wait()
                    return 0

                lax.fori_loop(0, 16, drain, 0)

                rows = iota + t * 16
                mrow = rows < K
                p = spat[pl.ds(t * 16, 16)]
                base = (p % 32) * 4
                for comp in range(4):
                    vals = plsc.load_gather(
                        lbuf, [iota, base + comp], mask=mrow) * SCALE
                    plsc.store_scatter(det, [rows * 6 + comp], vals,
                                       mask=mrow)
                return 0

            lax.fori_loop(0, TOPK_PAD // 16, bx_body, 0)

            pltpu.sync_copy(det, out_hbm.at[b])

    return k(cls_pred, loc_blocks, pmax)


def kernel(cls_pred, loc_pred):
    pmax = _pixel_max(cls_pred)
    loc_blocks = loc_pred.reshape(B, H * W * 4 // 128, 128)
    det = _sc_decode(cls_pred, loc_blocks, pmax)
    return det[:, :600].reshape(B, K, 6)


# speedup vs baseline: 1.1231x; 1.0162x over previous
import functools

import jax
import jax.numpy as jnp
from jax import lax
from jax.experimental import pallas as pl
from jax.experimental.pallas import tpu as pltpu
from jax.experimental.pallas import tpu_sc as plsc

B, H, W, C = 16, 128, 128, 80
P = H * W
NGRP = 1024
K = 100
SCALE = 4.0
PIX_CAP = 256
SURV_CAP = 256
TOPK_PAD = 112
CH = 32
HHI = 0x7F800000


def _pmax_body(x_ref, o_ref):
    o_ref[...] = jnp.max(x_ref[...], axis=3)


def _pixel_max(cls_pred):
    bh = 128
    return pl.pallas_call(
        _pmax_body,
        grid=(B, H // bh),
        in_specs=[pl.BlockSpec((1, bh, W, C), lambda b, t: (b, t, 0, 0))],
        out_specs=pl.BlockSpec((1, bh, W), lambda b, t: (b, t, 0)),
        out_shape=jax.ShapeDtypeStruct((B, H, W), jnp.float32),
    )(cls_pred)


def _iota16():
    return lax.iota(jnp.int32, 16)


def _extract_f32(ref, i):
    blk = ref[pl.ds((i // 16) * 16, 16)]
    sel = jnp.where(_iota16() == (i % 16), blk, jnp.float32(-3.0))
    return jnp.max(sel)


def _extract_i32(ref, i):
    blk = ref[pl.ds((i // 16) * 16, 16)]
    sel = jnp.where(_iota16() == (i % 16), blk, jnp.int32(-2147483647))
    return jnp.max(sel)


def _pcnt(mask):
    return jnp.max(plsc.all_reduce_population_count(mask))


def _sc_decode(cls_pred, loc_blocks, pmax):
    mesh = plsc.VectorSubcoreMesh(core_axis_name="c", subcore_axis_name="s")

    @functools.partial(
        pl.kernel,
        out_type=jax.ShapeDtypeStruct((B, 640), jnp.float32),
        mesh=mesh,
        compiler_params=pltpu.CompilerParams(needs_layout_passes=False),
        scratch_types=[
            pltpu.VMEM((H, W), jnp.float32),
            pltpu.VMEM((NGRP,), jnp.float32),
            pltpu.VMEM((PIX_CAP + 16,), jnp.int32),
            pltpu.VMEM((CH * 8, C), jnp.float32),
            pltpu.VMEM((SURV_CAP + 16,), jnp.float32),
            pltpu.VMEM((SURV_CAP + 16,), jnp.int32),
            pltpu.VMEM((TOPK_PAD,), jnp.int32),
            pltpu.VMEM((TOPK_PAD, 128), jnp.float32),
            pltpu.VMEM((640,), jnp.float32),
            pltpu.SemaphoreType.DMA,
            pltpu.SemaphoreType.DMA,
        ],
    )
    def k(cls_hbm, loc_hbm, pmax_hbm, out_hbm,
          pbuf, gmax, pix, cbuf, sval, sidx, spat, lbuf, det, gsem, lsem):
        c = lax.axis_index("c")
        s = lax.axis_index("s")

        @pl.when(s < 8)
        def _work():
            b = c * 8 + s
            iota = _iota16()

            pltpu.sync_copy(pmax_hbm.at[b], pbuf)

            def gm_body(i, _):
                r0 = i // 8
                col = (i % 8) * 16

                def inner(j, acc):
                    return jnp.maximum(acc, pbuf[r0 + j * 8, pl.ds(col, 16)])

                acc = lax.fori_loop(1, 16, inner, pbuf[r0, pl.ds(col, 16)])
                gmax[pl.ds(i * 16, 16)] = acc
                return 0

            lax.fori_loop(0, NGRP // 16, gm_body, 0)

            def bs_body(_, carry):
                lo, hi = carry
                mid = lo + (hi - lo) // 2
                tf = lax.bitcast_convert_type(mid, jnp.float32)

                def cnt(g, acc):
                    return acc + plsc.all_reduce_population_count(
                        gmax[pl.ds(g * 16, 16)] >= tf)

                csplat = lax.fori_loop(0, NGRP // 16, cnt,
                                       jnp.zeros((16,), jnp.int32))
                big = jnp.max(csplat) >= K
                return (jnp.where(big, mid, lo), jnp.where(big, hi, mid))

            lo, _hi = lax.fori_loop(0, 31, bs_body,
                                    (jnp.int32(0), jnp.int32(HHI)))
            tf2 = lax.bitcast_convert_type(lo, jnp.float32)

            def cp_body(q, off):
                m = pbuf[q // 8, pl.ds((q % 8) * 16, 16)] >= tf2
                offc = jnp.minimum(off, PIX_CAP)
                plsc.store_compressed(pix.at[pl.ds(offc, 16)],
                                      q * 16 + iota, mask=m)
                return offc + _pcnt(m)

            npix = lax.fori_loop(0, P // 16, cp_body, jnp.int32(0))
            npix = jnp.minimum(npix, PIX_CAP)

            def ch_body(ci, moff):
                def issue(q, _):
                    fi = ci * CH + q

                    @pl.when(fi < npix)
                    def _():
                        p = _extract_i32(pix, fi)
                        y = p // W
                        x8 = ((p % W) // 8) * 8
                        pltpu.async_copy(
                            cls_hbm.at[b, y, pl.ds(x8, 8)],
                            cbuf.at[pl.ds(q * 8, 8)], gsem)
                    return 0

                lax.fori_loop(0, CH, issue, 0)

                def drain(q, _):
                    fi = ci * CH + q

                    @pl.when(fi < npix)
                    def _():
                        p = _extract_i32(pix, fi)
                        y = p // W
                        x8 = ((p % W) // 8) * 8
                        pltpu.make_async_copy(
                            cls_hbm.at[b, y, pl.ds(x8, 8)],
                            cbuf.at[pl.ds(q * 8, 8)], gsem).wait()
                    return 0

                lax.fori_loop(0, CH, drain, 0)

                def filt(q, moff2):
                    fi = ci * CH + q

                    def skip(m3):
                        return m3

                    def do(m3):
                        p = _extract_i32(pix, fi)
                        row = q * 8 + (p % 8)

                        def cgrp(g, m4):
                            vals = cbuf[row, pl.ds(g * 16, 16)]
                            m = vals >= tf2
                            mc = jnp.minimum(m4, SURV_CAP)
                            plsc.store_compressed(
                                sval.at[pl.ds(mc, 16)], vals, mask=m)
                            plsc.store_compressed(
                                sidx.at[pl.ds(mc, 16)],
                                p * C + g * 16 + iota, mask=m)
                            return mc + _pcnt(m)

                        return lax.fori_loop(0, C // 16, cgrp, m3)

                    return lax.cond(fi < npix, do, skip, moff2)

                return lax.fori_loop(0, CH, filt, moff)

            mcnt = lax.fori_loop(0, (npix + CH - 1) // CH, ch_body,
                                 jnp.int32(0))
            mcnt = jnp.minimum(mcnt, SURV_CAP)

            sval[pl.ds(mcnt, 16)] = jnp.full((16,), -1.0, jnp.float32)
            sidx[pl.ds(mcnt, 16)] = jnp.zeros((16,), jnp.int32)

            spat[pl.ds(96, 16)] = jnp.zeros((16,), jnp.int32)

            nblk = (mcnt + 15) // 16

            def rank_body(i, _):
                vi = _extract_f32(sval, i)
                xi = _extract_i32(sidx, i)

                def inner(g, acc):
                    vj = sval[pl.ds(g * 16, 16)]
                    xj = sidx[pl.ds(g * 16, 16)]
                    m = (vj > vi) | ((vj == vi) & (xj < xi))
                    return acc + plsc.all_reduce_population_count(m)

                rank = jnp.max(lax.fori_loop(0, nblk, inner,
                                             jnp.zeros((16,), jnp.int32)))

                @pl.when(rank < K)
                def _():
                    lane0 = iota == 0
                    plsc.store_scatter(
                        det, [jnp.full((16,), rank * 6 + 4, jnp.int32)],
                        jnp.full((16,), vi, jnp.float32), mask=lane0)
                    plsc.store_scatter(
                        det, [jnp.full((16,), rank * 6 + 5, jnp.int32)],
                        jnp.full((16,), (xi % C).astype(jnp.float32),
                                 jnp.float32), mask=lane0)
                    plsc.store_scatter(
                        spat, [jnp.full((16,), rank, jnp.int32)],
                        jnp.full((16,), xi // C, jnp.int32), mask=lane0)
                    pltpu.async_copy(loc_hbm.at[b, (xi // C) // 32],
                                     lbuf.at[rank], lsem)
                return 0

            lax.fori_loop(0, mcnt, rank_body, 0)

            def pad_issue(r, _):
                pltpu.async_copy(loc_hbm.at[b, 0], lbuf.at[r], lsem)
                return 0

            lax.fori_loop(K, TOPK_PAD, pad_issue, 0)

            def bx_drain(r, _):
                p = _extract_i32(spat, r)
                pltpu.make_async_copy(loc_hbm.at[b, p // 32],
                                      lbuf.at[r], lsem).wait()
                return 0

            lax.fori_loop(0, TOPK_PAD, bx_drain, 0)

            def bx_body(t, _):
                rows = iota + t * 16
                mrow = rows < K
                p = spat[pl.ds(t * 16, 16)]
                base = (p % 32) * 4
                for comp in range(4):
                    vals = plsc.load_gather(
                        lbuf, [rows, base + comp], mask=mrow) * SCALE
                    plsc.store_scatter(det, [rows * 6 + comp], vals,
                                       mask=mrow)
                return 0

            lax.fori_loop(0, TOPK_PAD // 16, bx_body, 0)

            pltpu.sync_copy(det, out_hbm.at[b])

    return k(cls_pred, loc_blocks, pmax)


def kernel(cls_pred, loc_pred):
    pmax = _pixel_max(cls_pred)
    loc_blocks = loc_pred.reshape(B, H * W * 4 // 128, 128)
    det = _sc_decode(cls_pred, loc_blocks, pmax)
    return det[:, :600].reshape(B, K, 6)
